# baseline (speedup 1.0000x reference)
import jax
import jax.numpy as jnp
from jax import lax
from jax.experimental import pallas as pl
from jax.experimental.pallas import tpu as pltpu
from jax.experimental.pallas import tpu_sc as plsc

EMBED_DIM = 128
BATCH = 16384
NUM_CORES = 2
NUM_SUBCORES = 16
NUM_WORKERS = NUM_CORES * NUM_SUBCORES
BPW = BATCH // NUM_WORKERS
CHUNK = 128
NCHUNK = BPW // CHUNK
TOTAL = 2 * NCHUNK
NBUF = 4


def _gather_body(table_hbm, ov_hbm, sc_hbm, out_ov, out_sc,
                 idx0, idx1, idx2, idx3, rows0, rows1, rows2, rows3,
                 gsem0, gsem1, gsem2, gsem3, wsem0, wsem1, wsem2, wsem3):
    wid = lax.axis_index("s") * NUM_CORES + lax.axis_index("c")
    base = wid * BPW

    idx_bufs = (idx0, idx1, idx2, idx3)
    row_bufs = (rows0, rows1, rows2, rows3)
    gsems = (gsem0, gsem1, gsem2, gsem3)
    wsems = (wsem0, wsem1, wsem2, wsem3)

    def chunk_refs(k):
        if k < NCHUNK:
            return ov_hbm.at[pl.ds(base + k * CHUNK, CHUNK)], \
                   out_ov.at[pl.ds(base + k * CHUNK, CHUNK)]
        j = k - NCHUNK
        return sc_hbm.at[pl.ds(base + j * CHUNK, CHUNK)], \
               out_sc.at[pl.ds(base + j * CHUNK, CHUNK)]

    gathers = [None] * NBUF
    writes = [None] * NBUF
    for k in range(TOTAL):
        slot = k % NBUF
        if writes[slot] is not None:
            writes[slot].wait()
        src_ref, _ = chunk_refs(k)
        pltpu.sync_copy(src_ref, idx_bufs[slot])
        gathers[slot] = pltpu.async_copy(
            table_hbm.at[idx_bufs[slot]], row_bufs[slot], gsems[slot])
        if k >= NBUF - 1:
            pk = k - (NBUF - 1)
            pslot = pk % NBUF
            _, dst_ref = chunk_refs(pk)
            gathers[pslot].wait()
            writes[pslot] = pltpu.async_copy(
                row_bufs[pslot], dst_ref, wsems[pslot])
    for pk in range(TOTAL - (NBUF - 1), TOTAL):
        pslot = pk % NBUF
        _, dst_ref = chunk_refs(pk)
        gathers[pslot].wait()
        writes[pslot] = pltpu.async_copy(
            row_bufs[pslot], dst_ref, wsems[pslot])
    for w in writes:
        if w is not None:
            w.wait()


@jax.jit
def kernel(overlap, scene, embed_table):
    ov = overlap.astype(jnp.int32)
    sc = scene.astype(jnp.int32)
    out_sds = jax.ShapeDtypeStruct((BATCH, EMBED_DIM), jnp.float32)
    run = pl.kernel(
        _gather_body,
        out_type=(out_sds, out_sds),
        mesh=plsc.VectorSubcoreMesh(core_axis_name="c", subcore_axis_name="s"),
        scratch_types=(
            [pltpu.VMEM((CHUNK,), jnp.int32)] * NBUF
            + [pltpu.VMEM((CHUNK, EMBED_DIM), jnp.float32)] * NBUF
            + [pltpu.SemaphoreType.DMA] * (2 * NBUF)
        ),
    )
    return run(embed_table, ov, sc)


# speedup vs baseline: 1.0175x; 1.0175x over previous
import jax
import jax.numpy as jnp
from jax import lax
from jax.experimental import pallas as pl
from jax.experimental.pallas import tpu as pltpu
from jax.experimental.pallas import tpu_sc as plsc

EMBED_DIM = 128
BATCH = 16384
NUM_CORES = 2
NUM_SUBCORES = 16
NUM_WORKERS = NUM_CORES * NUM_SUBCORES
BPW = BATCH // NUM_WORKERS
CHUNK = 128
NCHUNK = BPW // CHUNK
TOTAL = 2 * NCHUNK
NBUF = 6


def _gather_body(table_hbm, ov_hbm, sc_hbm, out_ov, out_sc,
                 idx_all, rows0, rows1, rows2, rows3, rows4, rows5,
                 isem, gsem0, gsem1, gsem2, gsem3, gsem4, gsem5,
                 wsem0, wsem1, wsem2, wsem3, wsem4, wsem5):
    wid = lax.axis_index("s") * NUM_CORES + lax.axis_index("c")
    row0 = wid * NCHUNK
    base = wid * BPW

    row_bufs = (rows0, rows1, rows2, rows3, rows4, rows5)
    gsems = (gsem0, gsem1, gsem2, gsem3, gsem4, gsem5)
    wsems = (wsem0, wsem1, wsem2, wsem3, wsem4, wsem5)

    cp_i0 = pltpu.async_copy(
        ov_hbm.at[pl.ds(row0, NCHUNK)], idx_all.at[pl.ds(0, NCHUNK)], isem)
    cp_i1 = pltpu.async_copy(
        sc_hbm.at[pl.ds(row0, NCHUNK)], idx_all.at[pl.ds(NCHUNK, NCHUNK)],
        isem)
    cp_i0.wait()
    cp_i1.wait()

    def out_ref(k):
        if k < NCHUNK:
            return out_ov.at[pl.ds(base + k * CHUNK, CHUNK)]
        return out_sc.at[pl.ds(base + (k - NCHUNK) * CHUNK, CHUNK)]

    gathers = [None] * NBUF
    writes = [None] * NBUF
    for k in range(TOTAL):
        slot = k % NBUF
        if writes[slot] is not None:
            writes[slot].wait()
        gathers[slot] = pltpu.async_copy(
            table_hbm.at[idx_all.at[k]], row_bufs[slot], gsems[slot])
        if k >= NBUF - 1:
            pk = k - (NBUF - 1)
            pslot = pk % NBUF
            gathers[pslot].wait()
            writes[pslot] = pltpu.async_copy(
                row_bufs[pslot], out_ref(pk), wsems[pslot])
    for pk in range(max(0, TOTAL - (NBUF - 1)), TOTAL):
        pslot = pk % NBUF
        gathers[pslot].wait()
        writes[pslot] = pltpu.async_copy(
            row_bufs[pslot], out_ref(pk), wsems[pslot])
    for w in writes:
        if w is not None:
            w.wait()


@jax.jit
def kernel(overlap, scene, embed_table):
    ov = overlap.astype(jnp.int32).reshape(BATCH // CHUNK, CHUNK)
    sc = scene.astype(jnp.int32).reshape(BATCH // CHUNK, CHUNK)
    out_sds = jax.ShapeDtypeStruct((BATCH, EMBED_DIM), jnp.float32)
    run = pl.kernel(
        _gather_body,
        out_type=(out_sds, out_sds),
        mesh=plsc.VectorSubcoreMesh(core_axis_name="c", subcore_axis_name="s"),
        scratch_types=(
            [pltpu.VMEM((TOTAL, CHUNK), jnp.int32)]
            + [pltpu.VMEM((CHUNK, EMBED_DIM), jnp.float32)] * NBUF
            + [pltpu.SemaphoreType.DMA] * (2 * NBUF + 1)
        ),
    )
    return run(embed_table, ov, sc)


# speedup vs baseline: 1.2961x; 1.2739x over previous
import jax
import jax.numpy as jnp
from jax import lax
from jax.experimental import pallas as pl
from jax.experimental.pallas import tpu as pltpu
from jax.experimental.pallas import tpu_sc as plsc

EMBED_DIM = 128
BATCH = 16384
VOCAB_ROWS = 1489
NUM_CORES = 2
NUM_SUBCORES = 16
NUM_WORKERS = NUM_CORES * NUM_SUBCORES
BPW = BATCH // NUM_WORKERS
CHUNK = 128
NCHUNK = BPW // CHUNK
TOTAL = 2 * NCHUNK
NBUF = 6


TROWS = 96


def _gather_body(table_hbm, ov_hbm, sc_hbm, out_ov, out_sc,
                 table_sh, idx_all, rows0, rows1, rows2, rows3, rows4, rows5,
                 isem, tsem, gsem0, gsem1, gsem2, gsem3, gsem4, gsem5,
                 wsem0, wsem1, wsem2, wsem3, wsem4, wsem5):
    sid = lax.axis_index("s")
    wid = sid * NUM_CORES + lax.axis_index("c")
    row0 = wid * NCHUNK
    base = wid * BPW

    row_bufs = (rows0, rows1, rows2, rows3, rows4, rows5)
    gsems = (gsem0, gsem1, gsem2, gsem3, gsem4, gsem5)
    wsems = (wsem0, wsem1, wsem2, wsem3, wsem4, wsem5)

    tail = sid == NUM_SUBCORES - 1

    @pl.when(jnp.logical_not(tail))
    def _stage_main():
        pltpu.async_copy(
            table_hbm.at[pl.ds(sid * TROWS, TROWS)],
            table_sh.at[pl.ds(sid * TROWS, TROWS)], tsem).wait()

    @pl.when(tail)
    def _stage_tail():
        pltpu.async_copy(
            table_hbm.at[pl.ds(15 * TROWS, VOCAB_ROWS - 15 * TROWS)],
            table_sh.at[pl.ds(15 * TROWS, VOCAB_ROWS - 15 * TROWS)],
            tsem).wait()

    cp_i0 = pltpu.async_copy(
        ov_hbm.at[pl.ds(row0, NCHUNK)], idx_all.at[pl.ds(0, NCHUNK)], isem)
    cp_i1 = pltpu.async_copy(
        sc_hbm.at[pl.ds(row0, NCHUNK)], idx_all.at[pl.ds(NCHUNK, NCHUNK)],
        isem)
    cp_i0.wait()
    cp_i1.wait()
    plsc.subcore_barrier()

    def out_ref(k):
        if k < NCHUNK:
            return out_ov.at[pl.ds(base + k * CHUNK, CHUNK)]
        return out_sc.at[pl.ds(base + (k - NCHUNK) * CHUNK, CHUNK)]

    gathers = [None] * NBUF
    writes = [None] * NBUF
    for k in range(TOTAL):
        slot = k % NBUF
        if writes[slot] is not None:
            writes[slot].wait()
        gathers[slot] = pltpu.async_copy(
            table_sh.at[idx_all.at[k]], row_bufs[slot], gsems[slot])
        if k >= NBUF - 1:
            pk = k - (NBUF - 1)
            pslot = pk % NBUF
            gathers[pslot].wait()
            writes[pslot] = pltpu.async_copy(
                row_bufs[pslot], out_ref(pk), wsems[pslot])
    for pk in range(max(0, TOTAL - (NBUF - 1)), TOTAL):
        pslot = pk % NBUF
        gathers[pslot].wait()
        writes[pslot] = pltpu.async_copy(
            row_bufs[pslot], out_ref(pk), wsems[pslot])
    for w in writes:
        if w is not None:
            w.wait()


@jax.jit
def kernel(overlap, scene, embed_table):
    ov = overlap.astype(jnp.int32).reshape(BATCH // CHUNK, CHUNK)
    sc = scene.astype(jnp.int32).reshape(BATCH // CHUNK, CHUNK)
    out_sds = jax.ShapeDtypeStruct((BATCH, EMBED_DIM), jnp.float32)
    run = pl.kernel(
        _gather_body,
        out_type=(out_sds, out_sds),
        mesh=plsc.VectorSubcoreMesh(core_axis_name="c", subcore_axis_name="s"),
        scratch_types=(
            [pltpu.VMEM_SHARED((VOCAB_ROWS, EMBED_DIM), jnp.float32)]
            + [pltpu.VMEM((TOTAL, CHUNK), jnp.int32)]
            + [pltpu.VMEM((CHUNK, EMBED_DIM), jnp.float32)] * NBUF
            + [pltpu.SemaphoreType.DMA] * (2 * NBUF + 2)
        ),
    )
    return run(embed_table, ov, sc)
